# baseline (speedup 1.0000x reference)
import functools

import numpy as np
import jax
import jax.numpy as jnp
from jax import lax
from jax.experimental import pallas as pl
from jax.experimental.pallas import tpu as pltpu
from jax.experimental.pallas import tpu_sc as plsc

_N = 100000
_E = 1600000
_T = 48
_D = 32

_NPAD = 102400
_BN = 6400


def _conv_body(x_ref, pt_ref, y_ref):
    ts = lax.broadcasted_iota(jnp.int32, (_D, 1), 0).astype(jnp.float32) + 1.0
    logt = jnp.log(ts)
    p0 = pt_ref[0:1, :]
    p1 = pt_ref[1:2, :]
    s0 = 1.0 / (1.0 + jnp.exp(-p0))
    s1 = 1.0 / (1.0 + jnp.exp(-p1))
    a = 1.0 + (s0 * 0.25 + 0.005) * 10.0
    b = 0.1 + s1 * 1.2
    logk = (a - 1.0) * logt - ts / b
    k = jnp.exp(logk)
    k = k / (jnp.sum(k, axis=0, keepdims=True) + 1e-8)
    xblk = x_ref[...]
    xp = jnp.concatenate(
        [jnp.zeros((_D - 1, _BN), jnp.float32), xblk], axis=0)
    y = k[0:1, :] * xblk
    for d in range(1, _D):
        y = y + k[d:d + 1, :] * lax.slice_in_dim(xp, _D - 1 - d,
                                                 _D - 1 - d + _T, axis=0)
    y_ref[...] = y


def _conv(x, params_t):
    return pl.pallas_call(
        _conv_body,
        grid=(_NPAD // _BN,),
        in_specs=[
            pl.BlockSpec((_T, _BN), lambda i: (0, i)),
            pl.BlockSpec((2, _BN), lambda i: (0, i)),
        ],
        out_specs=pl.BlockSpec((_T, _BN), lambda i: (0, i)),
        out_shape=jax.ShapeDtypeStruct((_T, _NPAD), jnp.float32),
    )(x, params_t)


_NPART = 25000
_CB = 400
_EPT = _E // 16
_NCHUNK = _EPT // _CB
_NSUPER = _NCHUNK // 2
_CAP = 544
_BLK = 128
_ROWS = 512
_NBLK = _ROWS // _BLK

_mesh = plsc.VectorSubcoreMesh(core_axis_name="c", subcore_axis_name="s")


@functools.partial(
    pl.kernel,
    out_type=jax.ShapeDtypeStruct((_N, _T), jnp.float32),
    mesh=_mesh,
    compiler_params=pltpu.CompilerParams(needs_layout_passes=False,
                                         use_tc_tiling_on_sc=False),
    scratch_types=[
        pltpu.VMEM_SHARED((_NPART + 16, _T), jnp.float32),
        pltpu.VMEM((_CB,), jnp.int32),
        pltpu.VMEM((_CB,), jnp.int32),
        pltpu.VMEM((_CAP,), jnp.int32),
        pltpu.VMEM((_NBLK + 1, _BLK), jnp.int32),
        pltpu.VMEM((_ROWS, _T), jnp.float32),
        pltpu.VMEM((_CB,), jnp.int32),
        pltpu.VMEM((_CB,), jnp.int32),
        pltpu.VMEM((_CAP,), jnp.int32),
        pltpu.VMEM((_NBLK + 1, _BLK), jnp.int32),
        pltpu.VMEM((_ROWS, _T), jnp.float32),
        pltpu.SemaphoreType.DMA,
        pltpu.SemaphoreType.DMA,
        pltpu.SemaphoreType.DMA,
        pltpu.SemaphoreType.DMA,
        pltpu.SemaphoreType.DMA,
        pltpu.SemaphoreType.DMA,
    ],
)
def _route(yt, src_e, dst_e, out, accum,
           dstvA, srcvA, gidxA, sidx2A, rowsA,
           dstvB, srcvB, gidxB, sidx2B, rowsB,
           esemA, esemB, gsemA, gsemB, ssemA, ssemB):
    c = lax.axis_index("c")
    s = lax.axis_index("s")
    bufA = (dstvA, srcvA, gidxA, sidx2A, rowsA, esemA, gsemA, ssemA)
    bufB = (dstvB, srcvB, gidxB, sidx2B, rowsB, esemB, gsemB, ssemB)

    def edges_issue(ch, buf):
        dstv, srcv, _, _, _, esem, _, _ = buf
        base = s * _EPT + ch * _CB
        pltpu.async_copy(dst_e.at[pl.ds(base, _CB)], dstv, esem)
        pltpu.async_copy(src_e.at[pl.ds(base, _CB)], srcv, esem)

    def edges_wait(buf):
        dstv, srcv, _, _, _, esem, _, _ = buf
        pltpu.make_async_copy(dst_e.at[pl.ds(0, _CB)], dstv, esem).wait()
        pltpu.make_async_copy(src_e.at[pl.ds(0, _CB)], srcv, esem).wait()

    def gfire(buf, nblk):
        _, _, gidx, _, rows, _, gsem, _ = buf

        def go(j, _):
            pltpu.async_copy(yt.at[gidx.at[pl.ds(j * _BLK, _BLK)]],
                             rows.at[pl.ds(j * _BLK, _BLK)], gsem)
            return 0

        lax.fori_loop(0, nblk, go, 0)

    def gdrain(buf, nblk):
        _, _, gidx, _, rows, _, gsem, _ = buf

        def go(j, _):
            pltpu.make_async_copy(yt.at[gidx.at[pl.ds(j * _BLK, _BLK)]],
                                  rows.at[pl.ds(j * _BLK, _BLK)],
                                  gsem).wait()
            return 0

        lax.fori_loop(0, nblk, go, 0)

    def sfire(buf, nblk):
        _, _, _, sidx2, rows, _, _, ssem = buf

        def go(j, _):
            pltpu.async_copy(rows.at[pl.ds(j * _BLK, _BLK)],
                             accum.at[sidx2.at[j]], ssem, add=True)
            return 0

        lax.fori_loop(0, nblk, go, 0)

    def sdrain(buf, nblk):
        _, _, _, sidx2, rows, _, _, ssem = buf

        def go(j, _):
            pltpu.make_async_copy(rows.at[pl.ds(j * _BLK, _BLK)],
                                  accum.at[sidx2.at[j]], ssem).wait()
            return 0

        lax.fori_loop(0, nblk, go, 0)

    for p in range(2):
        lo = (2 * c + p) * _NPART
        trash = _NPART + s

        for k in range(2):
            idx = s + 16 * k

            @pl.when(idx < 25)
            def _():
                pltpu.sync_copy(yt.at[pl.ds(lo + idx * 1000, 1000)],
                                accum.at[pl.ds(idx * 1000, 1000)])
        plsc.subcore_barrier()

        def scan(buf):
            dstv, srcv, gidx, sidx2, _, _, _, _ = buf

            def scan_body(i, cnt):
                d = dstv[pl.ds(i * 16, 16)]
                sv = srcv[pl.ds(i * 16, 16)]
                m = jnp.logical_and(d >= lo, d < lo + _NPART)
                mi = jnp.where(m, 1, 0)
                pos = cnt + plsc.cumsum(mi) - 1
                plsc.store_scatter(gidx, [pos], sv, mask=m)
                plsc.store_scatter(sidx2, [pos >> 7, pos & 127], d - lo,
                                   mask=m)
                return cnt + jnp.sum(mi)

            cnt = lax.fori_loop(0, _CB // 16, scan_body, 0)
            zero16 = jnp.zeros((16,), jnp.int32)
            for k in range(_BLK // 16):
                ppos = cnt + k * 16 + lax.iota(jnp.int32, 16)
                gidx[pl.ds(cnt + k * 16, 16)] = zero16
                plsc.store_scatter(sidx2, [ppos >> 7, ppos & 127],
                                   zero16 + trash)
            return (cnt + _BLK - 1) // _BLK

        edges_issue(0, bufA)
        edges_issue(1, bufB)

        def super_body(k, carry):
            nblkA_p, nblkB_p = carry
            gdrain(bufB, nblkB_p)
            sfire(bufB, nblkB_p)
            sdrain(bufA, nblkA_p)
            edges_wait(bufA)
            nblkA = scan(bufA)

            @pl.when(k < _NSUPER - 1)
            def _():
                edges_issue(2 * k + 2, bufA)

            gfire(bufA, nblkA)
            sdrain(bufB, nblkB_p)
            edges_wait(bufB)
            nblkB = scan(bufB)

            @pl.when(k < _NSUPER - 1)
            def _():
                edges_issue(2 * k + 3, bufB)

            gfire(bufB, nblkB)
            gdrain(bufA, nblkA)
            sfire(bufA, nblkA)
            return (nblkA, nblkB)

        nblkA, nblkB = lax.fori_loop(0, _NSUPER, super_body, (0, 0))
        gdrain(bufB, nblkB)
        sfire(bufB, nblkB)
        sdrain(bufA, nblkA)
        sdrain(bufB, nblkB)
        plsc.subcore_barrier()
        for k in range(2):
            idx = s + 16 * k

            @pl.when(idx < 25)
            def _():
                pltpu.sync_copy(accum.at[pl.ds(idx * 1000, 1000)],
                                out.at[pl.ds(lo + idx * 1000, 1000)])
        plsc.subcore_barrier()


def kernel(x, edge_index, params):
    xpad = jnp.pad(x, ((0, 0), (0, _NPAD - _N)))
    ppad = jnp.pad(params.T, ((0, 0), (0, _NPAD - _N)))
    y = _conv(xpad, ppad)
    yt = y[:, :_N].T
    routed = _route(yt, edge_index[0], edge_index[1])
    return routed.T


# speedup vs baseline: 2.0767x; 2.0767x over previous
import functools

import numpy as np
import jax
import jax.numpy as jnp
from jax import lax
from jax.experimental import pallas as pl
from jax.experimental.pallas import tpu as pltpu
from jax.experimental.pallas import tpu_sc as plsc

_N = 100000
_E = 1600000
_T = 48
_D = 32

_NPAD = 102400
_BN = 6400


def _conv_body(x_ref, pt_ref, y_ref):
    ts = lax.broadcasted_iota(jnp.int32, (_D, 1), 0).astype(jnp.float32) + 1.0
    logt = jnp.log(ts)
    p0 = pt_ref[0:1, :]
    p1 = pt_ref[1:2, :]
    s0 = 1.0 / (1.0 + jnp.exp(-p0))
    s1 = 1.0 / (1.0 + jnp.exp(-p1))
    a = 1.0 + (s0 * 0.25 + 0.005) * 10.0
    b = 0.1 + s1 * 1.2
    logk = (a - 1.0) * logt - ts / b
    k = jnp.exp(logk)
    k = k / (jnp.sum(k, axis=0, keepdims=True) + 1e-8)
    xblk = x_ref[...]
    xp = jnp.concatenate(
        [jnp.zeros((_D - 1, _BN), jnp.float32), xblk], axis=0)
    y = k[0:1, :] * xblk
    for d in range(1, _D):
        y = y + k[d:d + 1, :] * lax.slice_in_dim(xp, _D - 1 - d,
                                                 _D - 1 - d + _T, axis=0)
    y_ref[...] = y


def _conv(x, params_t):
    return pl.pallas_call(
        _conv_body,
        grid=(_NPAD // _BN,),
        in_specs=[
            pl.BlockSpec((_T, _BN), lambda i: (0, i)),
            pl.BlockSpec((2, _BN), lambda i: (0, i)),
        ],
        out_specs=pl.BlockSpec((_T, _BN), lambda i: (0, i)),
        out_shape=jax.ShapeDtypeStruct((_T, _NPAD), jnp.float32),
    )(x, params_t)


_NPART = 25000
_EPT = 102400
_EPAD = 16 * _EPT
_CB = 512
_NCHUNK = _EPT // _CB
_NSUPER = _NCHUNK // 2
_BLK = 128
_NBLK = _CB // _BLK

_mesh = plsc.VectorSubcoreMesh(core_axis_name="c", subcore_axis_name="s")


@functools.partial(
    pl.kernel,
    out_type=jax.ShapeDtypeStruct((_N, _T), jnp.float32),
    mesh=_mesh,
    compiler_params=pltpu.CompilerParams(needs_layout_passes=False,
                                         use_tc_tiling_on_sc=False),
    scratch_types=[
        pltpu.VMEM_SHARED((_NPART + 16, _T), jnp.float32),
        pltpu.VMEM((_CB,), jnp.int32),
        pltpu.VMEM((_CB,), jnp.int32),
        pltpu.VMEM((_CB,), jnp.int32),
        pltpu.VMEM((_NBLK, _BLK), jnp.int32),
        pltpu.VMEM((_CB, _T), jnp.float32),
        pltpu.VMEM((_CB,), jnp.int32),
        pltpu.VMEM((_CB,), jnp.int32),
        pltpu.VMEM((_CB,), jnp.int32),
        pltpu.VMEM((_NBLK, _BLK), jnp.int32),
        pltpu.VMEM((_CB, _T), jnp.float32),
        pltpu.SemaphoreType.DMA,
        pltpu.SemaphoreType.DMA,
        pltpu.SemaphoreType.DMA,
        pltpu.SemaphoreType.DMA,
        pltpu.SemaphoreType.DMA,
        pltpu.SemaphoreType.DMA,
    ],
)
def _route(yt, src_e, dst_e, out, accum,
           dstvA, srcvA, gidxA, sidxA, rowsA,
           dstvB, srcvB, gidxB, sidxB, rowsB,
           esemA, esemB, gsemA, gsemB, ssemA, ssemB):
    c = lax.axis_index("c")
    s = lax.axis_index("s")
    bufA = (dstvA, srcvA, gidxA, sidxA, rowsA, esemA, gsemA, ssemA)
    bufB = (dstvB, srcvB, gidxB, sidxB, rowsB, esemB, gsemB, ssemB)

    def edges_issue(ch, buf):
        dstv, srcv, _, _, _, esem, _, _ = buf
        base = s * _EPT + ch * _CB
        pltpu.async_copy(dst_e.at[pl.ds(base, _CB)], dstv, esem)
        pltpu.async_copy(src_e.at[pl.ds(base, _CB)], srcv, esem)

    def edges_wait(buf):
        dstv, srcv, _, _, _, esem, _, _ = buf
        pltpu.make_async_copy(dst_e.at[pl.ds(0, _CB)], dstv, esem).wait()
        pltpu.make_async_copy(src_e.at[pl.ds(0, _CB)], srcv, esem).wait()

    def gfire(buf):
        _, _, gidx, _, rows, _, gsem, _ = buf
        for j in range(_NBLK):
            pltpu.async_copy(yt.at[gidx.at[pl.ds(j * _BLK, _BLK)]],
                             rows.at[pl.ds(j * _BLK, _BLK)], gsem)

    def gdrain(buf):
        _, _, gidx, _, rows, _, gsem, _ = buf
        for j in range(_NBLK):
            pltpu.make_async_copy(yt.at[gidx.at[pl.ds(j * _BLK, _BLK)]],
                                  rows.at[pl.ds(j * _BLK, _BLK)],
                                  gsem).wait()

    def sfire(buf):
        _, _, _, sidx, rows, _, _, ssem = buf
        for j in range(_NBLK):
            pltpu.async_copy(rows.at[pl.ds(j * _BLK, _BLK)],
                             accum.at[sidx.at[j]], ssem, add=True)

    def sdrain(buf):
        _, _, _, sidx, rows, _, _, ssem = buf
        for j in range(_NBLK):
            pltpu.make_async_copy(rows.at[pl.ds(j * _BLK, _BLK)],
                                  accum.at[sidx.at[j]], ssem).wait()

    for p in range(2):
        lo = (2 * c + p) * _NPART
        trash = _NPART + s

        for k in range(2):
            idx = s + 16 * k

            @pl.when(idx < 25)
            def _():
                pltpu.sync_copy(yt.at[pl.ds(lo + idx * 1000, 1000)],
                                accum.at[pl.ds(idx * 1000, 1000)])
        plsc.subcore_barrier()

        def scan(buf):
            dstv, srcv, gidx, sidx, _, _, _, _ = buf

            def scan_body(i, _):
                d = dstv[pl.ds(i * 16, 16)]
                gidx[pl.ds(i * 16, 16)] = srcv[pl.ds(i * 16, 16)]
                m = jnp.logical_and(d >= lo, d < lo + _NPART)
                sidx[i // 8, pl.ds((i % 8) * 16, 16)] = (
                    jnp.where(m, d - lo, trash))
                return 0

            lax.fori_loop(0, _CB // 16, scan_body, 0)

        edges_issue(0, bufA)
        edges_issue(1, bufB)
        edges_wait(bufA)
        scan(bufA)
        edges_issue(2, bufA)
        gfire(bufA)
        edges_wait(bufB)
        scan(bufB)
        edges_issue(3, bufB)
        gfire(bufB)
        gdrain(bufA)
        sfire(bufA)

        def super_body(k, carry):
            gdrain(bufB)
            sfire(bufB)
            sdrain(bufA)
            edges_wait(bufA)
            scan(bufA)

            @pl.when(k < _NSUPER - 1)
            def _():
                edges_issue(2 * k + 2, bufA)

            gfire(bufA)
            sdrain(bufB)
            edges_wait(bufB)
            scan(bufB)

            @pl.when(k < _NSUPER - 1)
            def _():
                edges_issue(2 * k + 3, bufB)

            gfire(bufB)
            gdrain(bufA)
            sfire(bufA)
            return 0

        lax.fori_loop(1, _NSUPER, super_body, 0)
        gdrain(bufB)
        sfire(bufB)
        sdrain(bufA)
        sdrain(bufB)
        plsc.subcore_barrier()
        for k in range(2):
            idx = s + 16 * k

            @pl.when(idx < 25)
            def _():
                pltpu.sync_copy(accum.at[pl.ds(idx * 1000, 1000)],
                                out.at[pl.ds(lo + idx * 1000, 1000)])
        plsc.subcore_barrier()


def kernel(x, edge_index, params):
    xpad = jnp.pad(x, ((0, 0), (0, _NPAD - _N)))
    ppad = jnp.pad(params.T, ((0, 0), (0, _NPAD - _N)))
    y = _conv(xpad, ppad)
    yt = y[:, :_N].T
    src_p = jnp.pad(edge_index[0], (0, _EPAD - _E))
    dst_p = jnp.pad(edge_index[1], (0, _EPAD - _E), constant_values=-1)
    routed = _route(yt, src_p, dst_p)
    return routed.T
